# baseline (speedup 1.0000x reference)
import functools

import jax
import jax.numpy as jnp
from jax import lax
from jax.experimental import pallas as pl
from jax.experimental.pallas import tpu as pltpu
from jax.experimental.pallas import tpu_sc as plsc

HIDDEN = 128
HEADS = 8
HEAD_DIM = HIDDEN // HEADS

_NODE_BLK = 1000
_EDGE_BLK = 1000
_MATH_BLK = 512
_C = 80


def _node_proj_body(x_ref, wq_ref, bq_ref, wk_ref, bk_ref, wv_ref, bv_ref,
                    q_ref, kv_ref):
    x = x_ref[...]
    q = jnp.dot(x, wq_ref[...], preferred_element_type=jnp.float32) + bq_ref[...]
    k = jnp.dot(x, wk_ref[...], preferred_element_type=jnp.float32) + bk_ref[...]
    v = jnp.dot(x, wv_ref[...], preferred_element_type=jnp.float32) + bv_ref[...]
    q_ref[...] = q
    uk = (lax.bitcast_convert_type(k, jnp.uint32) + jnp.uint32(0x8000)) >> 16
    uv = (lax.bitcast_convert_type(v, jnp.uint32) + jnp.uint32(0x8000)) >> 16
    kv_ref[...] = (uv << 16) | uk


def _node_proj(ns_flat, Wq, bq, Wk, bk, Wv, bv):
    n = ns_flat.shape[0]
    grid = (n // _NODE_BLK,)
    wspec = pl.BlockSpec((HIDDEN, HIDDEN), lambda i: (0, 0))
    bspec = pl.BlockSpec((1, HIDDEN), lambda i: (0, 0))
    return pl.pallas_call(
        _node_proj_body,
        grid=grid,
        in_specs=[pl.BlockSpec((_NODE_BLK, HIDDEN), lambda i: (i, 0)),
                  wspec, bspec, wspec, bspec, wspec, bspec],
        out_specs=[pl.BlockSpec((_NODE_BLK, HIDDEN), lambda i: (i, 0)),
                   pl.BlockSpec((_NODE_BLK, HIDDEN), lambda i: (i, 0))],
        out_shape=[jax.ShapeDtypeStruct((n, HIDDEN), jnp.float32),
                   jax.ShapeDtypeStruct((n, HIDDEN), jnp.uint32)],
    )(ns_flat, Wq, bq, Wk, bk, Wv, bv)


def _gather(b_arr, s_arr, d_arr, qt, kvt):
    e = b_arr.shape[0]
    n_nodes2 = qt.shape[0]
    n_nodes = n_nodes2 // 2
    n_per_w = e // 32
    chunks = n_per_w // _C
    mesh = plsc.VectorSubcoreMesh(core_axis_name="c", subcore_axis_name="s")

    idx_t = pltpu.VMEM((_C,), jnp.int32)
    qrow_t = pltpu.VMEM((_C, HIDDEN), jnp.float32)
    kvrow_t = pltpu.VMEM((_C, HIDDEN), jnp.uint32)
    sem_t = pltpu.SemaphoreType.DMA

    @functools.partial(
        pl.kernel, mesh=mesh,
        out_type=[jax.ShapeDtypeStruct((e, HIDDEN), jnp.float32),
                  jax.ShapeDtypeStruct((e, HIDDEN), jnp.uint32),
                  jax.ShapeDtypeStruct((e,), jnp.int32)],
        scratch_types=([idx_t] * 10 + [qrow_t, qrow_t, kvrow_t, kvrow_t]
                       + [sem_t] * 6),
    )
    def k(b_hbm, s_hbm, d_hbm, qt_hbm, kvt_hbm, qg_hbm, kvg_hbm, fs_hbm, *scr):
        b_v, s_v, d_v, fs_v, fd_v = (scr[0:2], scr[2:4], scr[4:6],
                                     scr[6:8], scr[8:10])
        qr, kvr = scr[10:12], scr[12:14]
        sem_i, sem_g, sem_s = scr[14:16], scr[16:18], scr[18:20]
        cid = lax.axis_index("c")
        sid = lax.axis_index("s")
        wid = sid * 2 + cid
        base0 = wid * n_per_w

        def fire_idx(p, ci):
            base = base0 + ci * _C
            pltpu.async_copy(b_hbm.at[pl.ds(base, _C)], b_v[p], sem_i[p])
            pltpu.async_copy(s_hbm.at[pl.ds(base, _C)], s_v[p], sem_i[p])
            pltpu.async_copy(d_hbm.at[pl.ds(base, _C)], d_v[p], sem_i[p])

        def wait_idx(p):
            for dst in (b_v[p], s_v[p], d_v[p]):
                pltpu.make_async_copy(b_hbm.at[pl.ds(0, _C)], dst,
                                      sem_i[p]).wait()

        def wait_st(p):
            pltpu.make_async_copy(qr[p], qg_hbm.at[pl.ds(0, _C)],
                                  sem_s[p]).wait()
            pltpu.make_async_copy(kvr[p], kvg_hbm.at[pl.ds(0, _C)],
                                  sem_s[p]).wait()
            pltpu.make_async_copy(fs_v[p], fs_hbm.at[pl.ds(0, _C)],
                                  sem_s[p]).wait()

        def chunk_body(i, p):
            @pl.when(i >= 2)
            def _():
                wait_st(p)
            wait_idx(p)
            for j in range(_C // 16):
                sl = pl.ds(j * 16, 16)
                bj = b_v[p][sl] * n_nodes
                fs_v[p][sl] = bj + s_v[p][sl]
                fd_v[p][sl] = bj + d_v[p][sl]
            pltpu.async_copy(qt_hbm.at[fs_v[p]], qr[p], sem_g[p])
            pltpu.async_copy(kvt_hbm.at[fd_v[p]], kvr[p], sem_g[p])
            fire_idx(p, jnp.minimum(i + 2, chunks - 1))
            pltpu.make_async_copy(qt_hbm.at[pl.ds(0, _C)], qr[p],
                                  sem_g[p]).wait()
            pltpu.make_async_copy(kvt_hbm.at[pl.ds(0, _C)], kvr[p],
                                  sem_g[p]).wait()
            base = base0 + i * _C
            pltpu.async_copy(qr[p], qg_hbm.at[pl.ds(base, _C)], sem_s[p])
            pltpu.async_copy(kvr[p], kvg_hbm.at[pl.ds(base, _C)], sem_s[p])
            pltpu.async_copy(fs_v[p], fs_hbm.at[pl.ds(base, _C)], sem_s[p])

        fire_idx(0, 0)
        fire_idx(1, 1)

        def body(i, carry):
            @pl.when(i % 2 == 0)
            def _():
                chunk_body(i, 0)

            @pl.when(i % 2 == 1)
            def _():
                chunk_body(i, 1)
            return carry

        lax.fori_loop(0, chunks, body, jnp.int32(0))
        for p in (0, 1):
            wait_st(p)
            wait_idx(p)

    return k(b_arr, s_arr, d_arr, qt, kvt)


def _edge_math_body(ev_ref, qg_ref, kvg_ref, wk_ref, bk_ref, wve_ref, bve_ref,
                    out_ref):
    ev = ev_ref[...]
    ke = jnp.dot(ev, wk_ref[...], preferred_element_type=jnp.float32) + bk_ref[...]
    ve = jnp.dot(ev, wve_ref[...], preferred_element_type=jnp.float32) + bve_ref[...]
    qg = qg_ref[...]
    kv32 = kvg_ref[...]
    k = lax.bitcast_convert_type(kv32 << 16, jnp.float32) + ke
    v = lax.bitcast_convert_type(kv32 & jnp.uint32(0xFFFF0000), jnp.float32) + ve
    x = qg * k
    r = lax.broadcasted_iota(jnp.int32, (HIDDEN, HEADS), 0) // HEAD_DIM
    c = lax.broadcasted_iota(jnp.int32, (HIDDEN, HEADS), 1)
    seg = (r == c).astype(jnp.float32)
    logits = jnp.dot(x, seg, preferred_element_type=jnp.float32) * (1.0 / 4.0)
    m = jnp.max(logits, axis=-1, keepdims=True)
    ex = jnp.exp(logits - m)
    a = ex / jnp.sum(ex, axis=-1, keepdims=True)
    r2 = lax.broadcasted_iota(jnp.int32, (HEADS, HIDDEN), 0)
    c2 = lax.broadcasted_iota(jnp.int32, (HEADS, HIDDEN), 1) // HEAD_DIM
    seg_t = (r2 == c2).astype(jnp.float32)
    aw = jnp.dot(a, seg_t, preferred_element_type=jnp.float32)
    out_ref[...] = aw * v


def _edge_math(ev, qg, kvg, Wk, bk, Wve, bve):
    e = qg.shape[0]
    grid = (e // _MATH_BLK,)
    espec = pl.BlockSpec((_MATH_BLK, HIDDEN), lambda i: (i, 0))
    wspec = pl.BlockSpec((HIDDEN, HIDDEN), lambda i: (0, 0))
    bspec = pl.BlockSpec((1, HIDDEN), lambda i: (0, 0))
    return pl.pallas_call(
        _edge_math_body,
        grid=grid,
        in_specs=[espec, espec, espec,
                  wspec, bspec, wspec, bspec],
        out_specs=espec,
        out_shape=jax.ShapeDtypeStruct((e, HIDDEN), jnp.float32),
    )(ev, qg, kvg, Wk, bk, Wve, bve)


def _scatter(contribs, fss, zeros, n_nodes):
    acc_rows = zeros.shape[0]
    dummy = n_nodes
    zrows = acc_rows // 16
    orows = (n_nodes // 16) // 8 * 8
    tail = n_nodes - 16 * orows
    mesh = plsc.VectorSubcoreMesh(core_axis_name="c", subcore_axis_name="s")

    total = sum(c.shape[0] for c in contribs)
    idx_t = pltpu.VMEM((_C,), jnp.int32)
    cbuf_t = pltpu.VMEM((_C, HIDDEN), jnp.float32)
    sem_t = pltpu.SemaphoreType.DMA

    @functools.partial(
        pl.kernel, mesh=mesh,
        out_type=jax.ShapeDtypeStruct((2 * n_nodes, HIDDEN), jnp.float32),
        scratch_types=([idx_t] * 4 + [cbuf_t, cbuf_t]
                       + [pltpu.VMEM_SHARED((acc_rows, HIDDEN), jnp.float32)]
                       + [sem_t] * 4),
    )
    def k(*args):
        narr = len(contribs)
        c_hbms = args[0:narr]
        fs_hbms = args[narr:2 * narr]
        zeros_hbm, out_hbm = args[2 * narr], args[2 * narr + 1]
        scr = args[2 * narr + 2:]
        fs_v, lidx_v, cbuf_v = scr[0:2], scr[2:4], scr[4:6]
        acc_sh = scr[6]
        sem_l, sem_a = scr[7:9], scr[9:11]
        cid = lax.axis_index("c")
        sid = lax.axis_index("s")
        lo = cid * n_nodes
        pltpu.sync_copy(zeros_hbm.at[pl.ds(sid * zrows, zrows)],
                        acc_sh.at[pl.ds(sid * zrows, zrows)])
        plsc.subcore_barrier()

        def run_pass(contrib_hbm, fs_hbm, e_a):
            n_per_s = e_a // 16
            chunks = n_per_s // _C
            base0 = sid * n_per_s

            def fire_ld(p, ci):
                base = base0 + ci * _C
                pltpu.async_copy(fs_hbm.at[pl.ds(base, _C)], fs_v[p], sem_l[p])
                pltpu.async_copy(contrib_hbm.at[pl.ds(base, _C)], cbuf_v[p],
                                 sem_l[p])

            def wait_ld(p):
                pltpu.make_async_copy(fs_hbm.at[pl.ds(0, _C)], fs_v[p],
                                      sem_l[p]).wait()
                pltpu.make_async_copy(contrib_hbm.at[pl.ds(0, _C)], cbuf_v[p],
                                      sem_l[p]).wait()

            def chunk_body(i, p):
                wait_ld(p)
                for j in range(_C // 16):
                    sl = pl.ds(j * 16, 16)
                    f = fs_v[p][sl]
                    own = (f >= lo) & (f < lo + n_nodes)
                    lidx_v[p][sl] = jnp.where(own, f - lo, dummy)
                pltpu.async_copy(cbuf_v[p], acc_sh.at[lidx_v[p]], sem_a[p],
                                 add=True)
                pltpu.make_async_copy(cbuf_v[p], acc_sh.at[pl.ds(0, _C)],
                                      sem_a[p]).wait()
                fire_ld(p, jnp.minimum(i + 2, chunks - 1))

            fire_ld(0, 0)
            fire_ld(1, 1)

            def body(i, carry):
                @pl.when(i % 2 == 0)
                def _():
                    chunk_body(i, 0)

                @pl.when(i % 2 == 1)
                def _():
                    chunk_body(i, 1)
                return carry

            lax.fori_loop(0, chunks, body, jnp.int32(0))
            for p in (0, 1):
                wait_ld(p)

        for c_hbm, f_hbm, carr in zip(c_hbms, fs_hbms, contribs):
            run_pass(c_hbm, f_hbm, carr.shape[0])
        plsc.subcore_barrier()
        pltpu.sync_copy(acc_sh.at[pl.ds(sid * orows, orows)],
                        out_hbm.at[pl.ds(cid * n_nodes + sid * orows, orows)])
        if tail:
            @pl.when(sid == 0)
            def _():
                pltpu.sync_copy(acc_sh.at[pl.ds(16 * orows, tail)],
                                out_hbm.at[pl.ds(cid * n_nodes + 16 * orows, tail)])

    return k(*contribs, *fss, zeros)


def kernel(node_states, edge_indices, edge_values, Wq, bq, Wk, bk, Wv, bv, Wve, bve):
    batch, n_nodes, hidden = node_states.shape
    ns_flat = node_states.reshape(batch * n_nodes, hidden)
    bq2, bk2, bv2, bve2 = (x.reshape(1, hidden) for x in (bq, bk, bv, bve))

    qt, kvt = _node_proj(ns_flat, Wq, bq2, Wk, bk2, Wv, bv2)

    b_arr = edge_indices[0]
    s_arr = edge_indices[1]
    d_arr = edge_indices[2]

    e = b_arr.shape[0]
    e1 = (e // 2) // (32 * _C) * (32 * _C)
    gathered = []
    for lo_, hi_ in ((0, e1), (e1, e)):
        gathered.append(_gather(b_arr[lo_:hi_], s_arr[lo_:hi_],
                                d_arr[lo_:hi_], qt, kvt))
    contribs = []
    for (qg, kvg, fs), (lo_, hi_) in zip(gathered, ((0, e1), (e1, e))):
        contribs.append(_edge_math(edge_values[lo_:hi_], qg, kvg,
                                   Wk, bk2, Wve, bve2))

    acc_rows = (n_nodes + 128) // 128 * 128
    zeros = jnp.zeros((acc_rows, hidden), jnp.float32)
    out_flat = _scatter(contribs, [g[2] for g in gathered], zeros, n_nodes)
    return out_flat.reshape(batch, n_nodes, hidden)


# speedup vs baseline: 1.0042x; 1.0042x over previous
import functools

import jax
import jax.numpy as jnp
from jax import lax
from jax.experimental import pallas as pl
from jax.experimental.pallas import tpu as pltpu
from jax.experimental.pallas import tpu_sc as plsc

HIDDEN = 128
HEADS = 8
HEAD_DIM = HIDDEN // HEADS

_NODE_BLK = 1000
_EDGE_BLK = 1000
_MATH_BLK = 512
_C = 80


def _node_proj_body(x_ref, wq_ref, bq_ref, wk_ref, bk_ref, wv_ref, bv_ref,
                    q_ref, kv_ref):
    x = x_ref[...]
    q = jnp.dot(x, wq_ref[...], preferred_element_type=jnp.float32) + bq_ref[...]
    k = jnp.dot(x, wk_ref[...], preferred_element_type=jnp.float32) + bk_ref[...]
    v = jnp.dot(x, wv_ref[...], preferred_element_type=jnp.float32) + bv_ref[...]
    q_ref[...] = q
    uk = (lax.bitcast_convert_type(k, jnp.uint32) + jnp.uint32(0x8000)) >> 16
    uv = (lax.bitcast_convert_type(v, jnp.uint32) + jnp.uint32(0x8000)) >> 16
    kv_ref[...] = (uv << 16) | uk


def _node_proj(ns_flat, Wq, bq, Wk, bk, Wv, bv):
    n = ns_flat.shape[0]
    grid = (n // _NODE_BLK,)
    wspec = pl.BlockSpec((HIDDEN, HIDDEN), lambda i: (0, 0))
    bspec = pl.BlockSpec((1, HIDDEN), lambda i: (0, 0))
    return pl.pallas_call(
        _node_proj_body,
        grid=grid,
        in_specs=[pl.BlockSpec((_NODE_BLK, HIDDEN), lambda i: (i, 0)),
                  wspec, bspec, wspec, bspec, wspec, bspec],
        out_specs=[pl.BlockSpec((_NODE_BLK, HIDDEN), lambda i: (i, 0)),
                   pl.BlockSpec((_NODE_BLK, HIDDEN), lambda i: (i, 0))],
        out_shape=[jax.ShapeDtypeStruct((n, HIDDEN), jnp.float32),
                   jax.ShapeDtypeStruct((n, HIDDEN), jnp.uint32)],
    )(ns_flat, Wq, bq, Wk, bk, Wv, bv)


def _gather(b_arr, s_arr, d_arr, qt, kvt):
    e = b_arr.shape[0]
    n_nodes2 = qt.shape[0]
    n_nodes = n_nodes2 // 2
    n_per_w = e // 32
    chunks = n_per_w // _C
    mesh = plsc.VectorSubcoreMesh(core_axis_name="c", subcore_axis_name="s")

    idx_t = pltpu.VMEM((_C,), jnp.int32)
    qrow_t = pltpu.VMEM((_C, HIDDEN), jnp.float32)
    kvrow_t = pltpu.VMEM((_C, HIDDEN), jnp.uint32)
    sem_t = pltpu.SemaphoreType.DMA

    @functools.partial(
        pl.kernel, mesh=mesh,
        out_type=[jax.ShapeDtypeStruct((e, HIDDEN), jnp.float32),
                  jax.ShapeDtypeStruct((e, HIDDEN), jnp.uint32),
                  jax.ShapeDtypeStruct((e,), jnp.int32)],
        scratch_types=([idx_t] * 10 + [qrow_t, qrow_t, kvrow_t, kvrow_t]
                       + [sem_t] * 6),
    )
    def k(b_hbm, s_hbm, d_hbm, qt_hbm, kvt_hbm, qg_hbm, kvg_hbm, fs_hbm, *scr):
        b_v, s_v, d_v, fs_v, fd_v = (scr[0:2], scr[2:4], scr[4:6],
                                     scr[6:8], scr[8:10])
        qr, kvr = scr[10:12], scr[12:14]
        sem_i, sem_g, sem_s = scr[14:16], scr[16:18], scr[18:20]
        cid = lax.axis_index("c")
        sid = lax.axis_index("s")
        wid = sid * 2 + cid
        base0 = wid * n_per_w

        def fire_idx(p, ci):
            base = base0 + ci * _C
            pltpu.async_copy(b_hbm.at[pl.ds(base, _C)], b_v[p], sem_i[p])
            pltpu.async_copy(s_hbm.at[pl.ds(base, _C)], s_v[p], sem_i[p])
            pltpu.async_copy(d_hbm.at[pl.ds(base, _C)], d_v[p], sem_i[p])

        def wait_idx(p):
            for dst in (b_v[p], s_v[p], d_v[p]):
                pltpu.make_async_copy(b_hbm.at[pl.ds(0, _C)], dst,
                                      sem_i[p]).wait()

        def wait_st(p):
            pltpu.make_async_copy(qr[p], qg_hbm.at[pl.ds(0, _C)],
                                  sem_s[p]).wait()
            pltpu.make_async_copy(kvr[p], kvg_hbm.at[pl.ds(0, _C)],
                                  sem_s[p]).wait()
            pltpu.make_async_copy(fs_v[p], fs_hbm.at[pl.ds(0, _C)],
                                  sem_s[p]).wait()

        def chunk_body(i, p):
            @pl.when(i >= 2)
            def _():
                wait_st(p)
            wait_idx(p)
            for j in range(_C // 16):
                sl = pl.ds(j * 16, 16)
                bj = b_v[p][sl] * n_nodes
                fs_v[p][sl] = bj + s_v[p][sl]
                fd_v[p][sl] = bj + d_v[p][sl]
            pltpu.async_copy(qt_hbm.at[fs_v[p]], qr[p], sem_g[p])
            pltpu.async_copy(kvt_hbm.at[fd_v[p]], kvr[p], sem_g[p])
            fire_idx(p, jnp.minimum(i + 2, chunks - 1))
            pltpu.make_async_copy(qt_hbm.at[pl.ds(0, _C)], qr[p],
                                  sem_g[p]).wait()
            pltpu.make_async_copy(kvt_hbm.at[pl.ds(0, _C)], kvr[p],
                                  sem_g[p]).wait()
            base = base0 + i * _C
            pltpu.async_copy(qr[p], qg_hbm.at[pl.ds(base, _C)], sem_s[p])
            pltpu.async_copy(kvr[p], kvg_hbm.at[pl.ds(base, _C)], sem_s[p])
            pltpu.async_copy(fs_v[p], fs_hbm.at[pl.ds(base, _C)], sem_s[p])

        fire_idx(0, 0)
        fire_idx(1, 1)

        def body(i, carry):
            @pl.when(i % 2 == 0)
            def _():
                chunk_body(i, 0)

            @pl.when(i % 2 == 1)
            def _():
                chunk_body(i, 1)
            return carry

        lax.fori_loop(0, chunks, body, jnp.int32(0))
        for p in (0, 1):
            wait_st(p)
            wait_idx(p)

    return k(b_arr, s_arr, d_arr, qt, kvt)


def _edge_math_body(ev_ref, qg_ref, kvg_ref, wk_ref, bk_ref, wve_ref, bve_ref,
                    out_ref):
    ev = ev_ref[...]
    ke = jnp.dot(ev, wk_ref[...], preferred_element_type=jnp.float32) + bk_ref[...]
    ve = jnp.dot(ev, wve_ref[...], preferred_element_type=jnp.float32) + bve_ref[...]
    qg = qg_ref[...]
    kv32 = kvg_ref[...]
    k = lax.bitcast_convert_type(kv32 << 16, jnp.float32) + ke
    v = lax.bitcast_convert_type(kv32 & jnp.uint32(0xFFFF0000), jnp.float32) + ve
    x = qg * k
    r = lax.broadcasted_iota(jnp.int32, (HIDDEN, HEADS), 0) // HEAD_DIM
    c = lax.broadcasted_iota(jnp.int32, (HIDDEN, HEADS), 1)
    seg = (r == c).astype(jnp.float32)
    logits = jnp.dot(x, seg, preferred_element_type=jnp.float32) * (1.0 / 4.0)
    m = jnp.max(logits, axis=-1, keepdims=True)
    ex = jnp.exp(logits - m)
    a = ex / jnp.sum(ex, axis=-1, keepdims=True)
    r2 = lax.broadcasted_iota(jnp.int32, (HEADS, HIDDEN), 0)
    c2 = lax.broadcasted_iota(jnp.int32, (HEADS, HIDDEN), 1) // HEAD_DIM
    seg_t = (r2 == c2).astype(jnp.float32)
    aw = jnp.dot(a, seg_t, preferred_element_type=jnp.float32)
    out_ref[...] = aw * v


def _edge_math(ev, qg, kvg, Wk, bk, Wve, bve):
    e = qg.shape[0]
    grid = (e // _MATH_BLK,)
    espec = pl.BlockSpec((_MATH_BLK, HIDDEN), lambda i: (i, 0))
    wspec = pl.BlockSpec((HIDDEN, HIDDEN), lambda i: (0, 0))
    bspec = pl.BlockSpec((1, HIDDEN), lambda i: (0, 0))
    return pl.pallas_call(
        _edge_math_body,
        grid=grid,
        in_specs=[espec, espec, espec,
                  wspec, bspec, wspec, bspec],
        out_specs=espec,
        out_shape=jax.ShapeDtypeStruct((e, HIDDEN), jnp.float32),
    )(ev, qg, kvg, Wk, bk, Wve, bve)


def _scatter(contrib, fs, zeros, n_nodes):
    e = contrib.shape[0]
    acc_rows = zeros.shape[0]
    dummy = n_nodes
    n_per_s = e // 16
    zrows = acc_rows // 16
    orows = (n_nodes // 16) // 8 * 8
    tail = n_nodes - 16 * orows
    mesh = plsc.VectorSubcoreMesh(core_axis_name="c", subcore_axis_name="s")

    chunks = n_per_s // _C
    idx_t = pltpu.VMEM((_C,), jnp.int32)
    cbuf_t = pltpu.VMEM((_C, HIDDEN), jnp.float32)
    sem_t = pltpu.SemaphoreType.DMA

    @functools.partial(
        pl.kernel, mesh=mesh,
        out_type=jax.ShapeDtypeStruct((2 * n_nodes, HIDDEN), jnp.float32),
        scratch_types=([idx_t] * 8 + [cbuf_t] * 4
                       + [pltpu.VMEM_SHARED((acc_rows, HIDDEN), jnp.float32)]
                       + [sem_t] * 8),
    )
    def k(contrib_hbm, fs_hbm, zeros_hbm, out_hbm, *scr):
        fs_v, lidx_v, cbuf_v = scr[0:4], scr[4:8], scr[8:12]
        acc_sh = scr[12]
        sem_l, sem_a = scr[13:17], scr[17:21]
        cid = lax.axis_index("c")
        sid = lax.axis_index("s")
        lo = cid * n_nodes
        pltpu.sync_copy(zeros_hbm.at[pl.ds(sid * zrows, zrows)],
                        acc_sh.at[pl.ds(sid * zrows, zrows)])
        plsc.subcore_barrier()
        base0 = sid * n_per_s

        def fire_ld(p, ci):
            base = base0 + ci * _C
            pltpu.async_copy(fs_hbm.at[pl.ds(base, _C)], fs_v[p], sem_l[p])
            pltpu.async_copy(contrib_hbm.at[pl.ds(base, _C)], cbuf_v[p],
                             sem_l[p])

        def wait_ld(p):
            pltpu.make_async_copy(fs_hbm.at[pl.ds(0, _C)], fs_v[p],
                                  sem_l[p]).wait()
            pltpu.make_async_copy(contrib_hbm.at[pl.ds(0, _C)], cbuf_v[p],
                                  sem_l[p]).wait()

        def wait_add(p):
            pltpu.make_async_copy(cbuf_v[p], acc_sh.at[pl.ds(0, _C)],
                                  sem_a[p]).wait()

        def chunk_body(i, p):
            wait_ld(p)
            for j in range(_C // 16):
                sl = pl.ds(j * 16, 16)
                f = fs_v[p][sl]
                own = (f >= lo) & (f < lo + n_nodes)
                lidx_v[p][sl] = jnp.where(own, f - lo, dummy)
            pltpu.async_copy(cbuf_v[p], acc_sh.at[lidx_v[p]], sem_a[p],
                             add=True)
            q = (p + 2) % 4

            @pl.when(i >= 2)
            def _():
                wait_add(q)
            fire_ld(q, jnp.minimum(i + 2, chunks - 1))

        fire_ld(0, 0)
        fire_ld(1, 1)

        def body(i, carry):
            for p in range(4):
                @pl.when(i % 4 == p)
                def _(p=p):
                    chunk_body(i, p)
            return carry

        lax.fori_loop(0, chunks, body, jnp.int32(0))
        for p in ((chunks - 2) % 4, (chunks - 1) % 4):
            wait_add(p)
        for p in (chunks % 4, (chunks + 1) % 4):
            wait_ld(p)
        plsc.subcore_barrier()
        pltpu.sync_copy(acc_sh.at[pl.ds(sid * orows, orows)],
                        out_hbm.at[pl.ds(cid * n_nodes + sid * orows, orows)])
        if tail:
            @pl.when(sid == 0)
            def _():
                pltpu.sync_copy(acc_sh.at[pl.ds(16 * orows, tail)],
                                out_hbm.at[pl.ds(cid * n_nodes + 16 * orows, tail)])

    return k(contrib, fs, zeros)


def kernel(node_states, edge_indices, edge_values, Wq, bq, Wk, bk, Wv, bv, Wve, bve):
    batch, n_nodes, hidden = node_states.shape
    ns_flat = node_states.reshape(batch * n_nodes, hidden)
    bq2, bk2, bv2, bve2 = (x.reshape(1, hidden) for x in (bq, bk, bv, bve))

    qt, kvt = _node_proj(ns_flat, Wq, bq2, Wk, bk2, Wv, bv2)

    b_arr = edge_indices[0]
    s_arr = edge_indices[1]
    d_arr = edge_indices[2]
    qg, kvg, fs = _gather(b_arr, s_arr, d_arr, qt, kvt)

    contrib = _edge_math(edge_values, qg, kvg, Wk, bk2, Wve, bve2)

    acc_rows = (n_nodes + 128) // 128 * 128
    zeros = jnp.zeros((acc_rows, hidden), jnp.float32)
    out_flat = _scatter(contrib, fs, zeros, n_nodes)
    return out_flat.reshape(batch, n_nodes, hidden)
